# baseline (speedup 1.0000x reference)
import jax
import jax.numpy as jnp
import numpy as np
from jax import lax
from jax.experimental import pallas as pl
from jax.experimental.pallas import tpu as pltpu
from jax.experimental.pallas import tpu_sc as plsc

_K = 8
_RADIUS = 100.0
_N = 8192
_M = 16384
_NW = 32
_QPW = _N // _NW
_L = 16
_MV = _M // _L
_SEGV = 8
_SEG = _MV // _SEGV
_Q = 4
_BIG = np.float32(3.0e38)
_MU = np.float32(1e-3)


def _knn_body(qx_h, qy_h, qz_h, qbx_h, qby_h, qbz_h,
              px_h, py_h, pz_h, pxd_h, pyd_h, pzd_h,
              od2_h, oidx_h,
              bx, by, bz, p2b, qxv, qyv, qzv, qbxv, qbyv, qbzv,
              segm, od2v, oidxv):
    wid = lax.axis_index("s") * 2 + lax.axis_index("c")
    base = wid * _QPW

    pltpu.sync_copy(px_h, bx)
    pltpu.sync_copy(py_h, by)
    pltpu.sync_copy(pz_h, bz)

    @plsc.parallel_loop(0, _MV, unroll=4)
    def p2_step(j):
        pxv = bx[pl.ds(j * _L, _L)]
        pyv = by[pl.ds(j * _L, _L)]
        pzv = bz[pl.ds(j * _L, _L)]
        p2b[pl.ds(j * _L, _L)] = (pxv * pxv + pyv * pyv) + pzv * pzv

    pltpu.sync_copy(pxd_h, bx)
    pltpu.sync_copy(pyd_h, by)
    pltpu.sync_copy(pzd_h, bz)

    pltpu.sync_copy(qx_h.at[pl.ds(base, _QPW)], qxv)
    pltpu.sync_copy(qy_h.at[pl.ds(base, _QPW)], qyv)
    pltpu.sync_copy(qz_h.at[pl.ds(base, _QPW)], qzv)
    pltpu.sync_copy(qbx_h.at[pl.ds(base, _QPW)], qbxv)
    pltpu.sync_copy(qby_h.at[pl.ds(base, _QPW)], qbyv)
    pltpu.sync_copy(qbz_h.at[pl.ds(base, _QPW)], qbzv)

    lane = lax.iota(jnp.int32, _L)
    big_vec = jnp.full((_L,), _BIG, jnp.float32)
    zero_idx = jnp.zeros((_L,), jnp.int32)

    def _qp2_at(off, qbx, qby, qbz):
        pxv = bx[pl.ds(off, _L)]
        pyv = by[pl.ds(off, _L)]
        pzv = bz[pl.ds(off, _L)]
        return (qbx * pxv + qby * pyv) + qbz * pzv

    def _sprime_at(off, qbx, qby, qbz):
        p2v = p2b[pl.ds(off, _L)]
        return p2v + _qp2_at(off, qbx, qby, qbz)

    def _d2_at(off, q2, qbx, qby, qbz):
        p2v = p2b[pl.ds(off, _L)]
        return jnp.maximum((q2 + p2v) + _qp2_at(off, qbx, qby, qbz), 0.0)

    def _any(mask):
        return plsc.all_reduce_population_count(mask)[0] > 0

    def per_block(blk, _):
        q0 = blk * _L
        qxg = qxv[pl.ds(q0, _L)]
        qyg = qyv[pl.ds(q0, _L)]
        qzg = qzv[pl.ds(q0, _L)]
        qbxg = qbxv[pl.ds(q0, _L)]
        qbyg = qbyv[pl.ds(q0, _L)]
        qbzg = qbzv[pl.ds(q0, _L)]
        for sub in range(_L // _Q):
            qs = []
            for k in range(_Q):
                j = sub * _Q + k
                qx, qy, qz = qxg[j], qyg[j], qzg[j]
                q2 = (qx * qx + qy * qy) + qz * qz
                qs.append((q2, qbxg[j], qbyg[j], qbzg[j]))

            @plsc.parallel_loop(0, _SEG, carry=(big_vec,) * _Q, unroll=2)
            def seg_min(s, gacc):
                ms = [big_vec] * _Q
                for v in range(_SEGV):
                    off = s * (_SEGV * _L) + v * _L
                    for k in range(_Q):
                        _, qbx, qby, qbz = qs[k]
                        ms[k] = jnp.minimum(
                            ms[k], _sprime_at(off, qbx, qby, qbz))
                for k in range(_Q):
                    segm[k, s, :] = ms[k]
                return tuple(jnp.minimum(gacc[k], ms[k])
                             for k in range(_Q))

            gs = seg_min

            for k in range(_Q):
                q2, qbx, qby, qbz = qs[k]
                t = jnp.sort(gs[k])[_K - 1]
                ts = t + (_MU + _MU)
                b = (t + q2) + _MU

                def merge(args):
                    R, Ri, mask, d2, ci = args
                    C = jnp.where(mask, d2, _BIG)
                    Cs, Cis = plsc.sort_key_val(C, ci, descending=True)
                    take = Cs < R
                    Lo = jnp.where(take, Cs, R)
                    Loi = jnp.where(take, Cis, Ri)
                    R2, Ri2 = plsc.sort_key_val(Lo, Loi)
                    return (R2, Ri2)

                def rescan(s, R, Ri, k=k):
                    for v in range(_SEGV):
                        off = s * (_SEGV * _L) + v * _L
                        d2 = _d2_at(off, q2, qbx, qby, qbz)
                        mask = d2 <= b
                        ci = off + lane
                        R, Ri = lax.cond(
                            _any(mask), merge,
                            lambda a: (a[0], a[1]),
                            (R, Ri, mask, d2, ci))
                    return R, Ri

                def seg_pair(sp, carry, k=k):
                    R, Ri = carry
                    s0 = sp + sp
                    m0 = segm[k, s0, :]
                    m1 = segm[k, s0 + 1, :]

                    def check_pair(args):
                        R, Ri = args
                        R, Ri = lax.cond(
                            _any(m0 <= ts),
                            lambda a: rescan(s0, a[0], a[1]),
                            lambda a: a, (R, Ri))
                        R, Ri = lax.cond(
                            _any(m1 <= ts),
                            lambda a: rescan(s0 + 1, a[0], a[1]),
                            lambda a: a, (R, Ri))
                        return R, Ri

                    return lax.cond(
                        _any(jnp.minimum(m0, m1) <= ts),
                        check_pair, lambda a: a, (R, Ri))

                R, Ri = lax.fori_loop(0, _SEG // 2, seg_pair,
                                      (big_vec, zero_idx))
                qi = q0 + sub * _Q + k
                od2v[qi, :] = R
                oidxv[qi, :] = Ri
        return 0

    lax.fori_loop(0, _QPW // _L, per_block, 0)

    pltpu.sync_copy(od2v, od2_h.at[pl.ds(base, _QPW)])
    pltpu.sync_copy(oidxv, oidx_h.at[pl.ds(base, _QPW)])


@jax.jit
def _sc_knn(qx, qy, qz, qbx, qby, qbz, px, py, pz, pxd, pyd, pzd):
    mesh = plsc.VectorSubcoreMesh(core_axis_name="c", subcore_axis_name="s")
    f = pl.kernel(
        _knn_body,
        mesh=mesh,
        compiler_params=pltpu.CompilerParams(
            needs_layout_passes=False, use_tc_tiling_on_sc=False),
        out_type=[
            jax.ShapeDtypeStruct((_N, _L), jnp.float32),
            jax.ShapeDtypeStruct((_N, _L), jnp.int32),
        ],
        scratch_types=[
            pltpu.VMEM((_M,), jnp.float32),
            pltpu.VMEM((_M,), jnp.float32),
            pltpu.VMEM((_M,), jnp.float32),
            pltpu.VMEM((_M,), jnp.float32),
            pltpu.VMEM((_QPW,), jnp.float32),
            pltpu.VMEM((_QPW,), jnp.float32),
            pltpu.VMEM((_QPW,), jnp.float32),
            pltpu.VMEM((_QPW,), jnp.float32),
            pltpu.VMEM((_QPW,), jnp.float32),
            pltpu.VMEM((_QPW,), jnp.float32),
            pltpu.VMEM((_Q, _SEG, _L), jnp.float32),
            pltpu.VMEM((_QPW, _L), jnp.float32),
            pltpu.VMEM((_QPW, _L), jnp.int32),
        ],
    )
    return f(qx, qy, qz, qbx, qby, qbz, px, py, pz, pxd, pyd, pzd)


def kernel(queries, points):
    q = queries[0]
    p = points[0]
    qb = lax.reduce_precision(q, 8, 7)
    pb = lax.reduce_precision(p, 8, 7)
    pd = pb * np.float32(-2.0)
    d2_16, idx_16 = _sc_knn(
        q[:, 0], q[:, 1], q[:, 2], qb[:, 0], qb[:, 1], qb[:, 2],
        p[:, 0], p[:, 1], p[:, 2], pd[:, 0], pd[:, 1], pd[:, 2])
    d2k = d2_16[:, :_K]
    idx = idx_16[:, :_K]
    valid = d2k <= (_RADIUS * _RADIUS)
    indices = jnp.where(valid, idx, -1)
    dist = jnp.sqrt(d2k)
    weights = 1.0 / (dist + 1e-7)
    weights = weights / jnp.sum(weights, axis=-1, keepdims=True)
    return (indices[None].astype(jnp.int64), weights[None], dist[None])


# speedup vs baseline: 1.8718x; 1.8718x over previous
import jax
import jax.numpy as jnp
import numpy as np
from jax import lax
from jax.experimental import pallas as pl
from jax.experimental.pallas import tpu as pltpu
from jax.experimental.pallas import tpu_sc as plsc

_K = 8
_RADIUS = 100.0
_N = 8192
_M = 16384
_NW = 32
_QPW = _N // _NW
_L = 16
_MV = _M // _L
_SEGV = 8
_SEG = _MV // _SEGV
_Q = 4
_BIG = np.float32(3.0e38)
_MU = np.float32(1e-3)


def _knn_body(qx_h, qy_h, qz_h, qbx_h, qby_h, qbz_h,
              px_h, py_h, pz_h, pxd_h, pyd_h, pzd_h,
              od2_h, oidx_h,
              bx, by, bz, p2b, qxv, qyv, qzv, qbxv, qbyv, qbzv,
              segm, od2v, oidxv):
    wid = lax.axis_index("s") * 2 + lax.axis_index("c")
    base = wid * _QPW

    pltpu.sync_copy(px_h, bx)
    pltpu.sync_copy(py_h, by)
    pltpu.sync_copy(pz_h, bz)

    def p2_step(j, _):
        pxv = bx[pl.ds(j * _L, _L)]
        pyv = by[pl.ds(j * _L, _L)]
        pzv = bz[pl.ds(j * _L, _L)]
        p2b[pl.ds(j * _L, _L)] = (pxv * pxv + pyv * pyv) + pzv * pzv
        return 0

    lax.fori_loop(0, _MV, p2_step, 0, unroll=4)

    pltpu.sync_copy(pxd_h, bx)
    pltpu.sync_copy(pyd_h, by)
    pltpu.sync_copy(pzd_h, bz)

    pltpu.sync_copy(qx_h.at[pl.ds(base, _QPW)], qxv)
    pltpu.sync_copy(qy_h.at[pl.ds(base, _QPW)], qyv)
    pltpu.sync_copy(qz_h.at[pl.ds(base, _QPW)], qzv)
    pltpu.sync_copy(qbx_h.at[pl.ds(base, _QPW)], qbxv)
    pltpu.sync_copy(qby_h.at[pl.ds(base, _QPW)], qbyv)
    pltpu.sync_copy(qbz_h.at[pl.ds(base, _QPW)], qbzv)

    lane = lax.iota(jnp.int32, _L)
    big_vec = jnp.full((_L,), _BIG, jnp.float32)
    zero_idx = jnp.zeros((_L,), jnp.int32)

    def _qp2_at(off, qbx, qby, qbz):
        pxv = bx[pl.ds(off, _L)]
        pyv = by[pl.ds(off, _L)]
        pzv = bz[pl.ds(off, _L)]
        return (qbx * pxv + qby * pyv) + qbz * pzv

    def _sprime_at(off, qbx, qby, qbz):
        p2v = p2b[pl.ds(off, _L)]
        return p2v + _qp2_at(off, qbx, qby, qbz)

    def _d2_at(off, q2, qbx, qby, qbz):
        p2v = p2b[pl.ds(off, _L)]
        return jnp.maximum((q2 + p2v) + _qp2_at(off, qbx, qby, qbz), 0.0)

    def _any(mask):
        return plsc.all_reduce_population_count(mask)[0] > 0

    def per_block(blk, _):
        q0 = blk * _L
        qxg = qxv[pl.ds(q0, _L)]
        qyg = qyv[pl.ds(q0, _L)]
        qzg = qzv[pl.ds(q0, _L)]
        qbxg = qbxv[pl.ds(q0, _L)]
        qbyg = qbyv[pl.ds(q0, _L)]
        qbzg = qbzv[pl.ds(q0, _L)]
        for sub in range(_L // _Q):
            qs = []
            for k in range(_Q):
                j = sub * _Q + k
                qx, qy, qz = qxg[j], qyg[j], qzg[j]
                q2 = (qx * qx + qy * qy) + qz * qz
                qs.append((q2, qbxg[j], qbyg[j], qbzg[j]))

            def seg_min(s, gacc):
                ms = [big_vec] * _Q
                for v in range(_SEGV):
                    off = s * (_SEGV * _L) + v * _L
                    for k in range(_Q):
                        _, qbx, qby, qbz = qs[k]
                        ms[k] = jnp.minimum(
                            ms[k], _sprime_at(off, qbx, qby, qbz))
                for k in range(_Q):
                    segm[k, s, :] = ms[k]
                return tuple(jnp.minimum(gacc[k], ms[k])
                             for k in range(_Q))

            gs = lax.fori_loop(0, _SEG, seg_min, (big_vec,) * _Q,
                               unroll=4)

            for k in range(_Q):
                q2, qbx, qby, qbz = qs[k]
                t = jnp.sort(gs[k])[_K - 1]
                ts = t + (_MU + _MU)
                b = (t + q2) + _MU

                def merge(args):
                    R, Ri, mask, d2, ci = args
                    C = jnp.where(mask, d2, _BIG)
                    Cs, Cis = plsc.sort_key_val(C, ci, descending=True)
                    take = Cs < R
                    Lo = jnp.where(take, Cs, R)
                    Loi = jnp.where(take, Cis, Ri)
                    R2, Ri2 = plsc.sort_key_val(Lo, Loi)
                    return (R2, Ri2)

                def rescan(s, R, Ri, k=k):
                    for v in range(_SEGV):
                        off = s * (_SEGV * _L) + v * _L
                        d2 = _d2_at(off, q2, qbx, qby, qbz)
                        mask = d2 <= b
                        ci = off + lane
                        R, Ri = lax.cond(
                            _any(mask), merge,
                            lambda a: (a[0], a[1]),
                            (R, Ri, mask, d2, ci))
                    return R, Ri

                def seg_pair(sp, carry, k=k):
                    R, Ri = carry
                    s0 = sp + sp
                    m0 = segm[k, s0, :]
                    m1 = segm[k, s0 + 1, :]

                    def check_pair(args):
                        R, Ri = args
                        R, Ri = lax.cond(
                            _any(m0 <= ts),
                            lambda a: rescan(s0, a[0], a[1]),
                            lambda a: a, (R, Ri))
                        R, Ri = lax.cond(
                            _any(m1 <= ts),
                            lambda a: rescan(s0 + 1, a[0], a[1]),
                            lambda a: a, (R, Ri))
                        return R, Ri

                    return lax.cond(
                        _any(jnp.minimum(m0, m1) <= ts),
                        check_pair, lambda a: a, (R, Ri))

                R, Ri = lax.fori_loop(0, _SEG // 2, seg_pair,
                                      (big_vec, zero_idx))
                qi = q0 + sub * _Q + k
                od2v[qi, :] = R
                oidxv[qi, :] = Ri
        return 0

    lax.fori_loop(0, _QPW // _L, per_block, 0)

    pltpu.sync_copy(od2v, od2_h.at[pl.ds(base, _QPW)])
    pltpu.sync_copy(oidxv, oidx_h.at[pl.ds(base, _QPW)])


@jax.jit
def _sc_knn(qx, qy, qz, qbx, qby, qbz, px, py, pz, pxd, pyd, pzd):
    mesh = plsc.VectorSubcoreMesh(core_axis_name="c", subcore_axis_name="s")
    f = pl.kernel(
        _knn_body,
        mesh=mesh,
        compiler_params=pltpu.CompilerParams(
            needs_layout_passes=False, use_tc_tiling_on_sc=False),
        out_type=[
            jax.ShapeDtypeStruct((_N, _L), jnp.float32),
            jax.ShapeDtypeStruct((_N, _L), jnp.int32),
        ],
        scratch_types=[
            pltpu.VMEM((_M,), jnp.float32),
            pltpu.VMEM((_M,), jnp.float32),
            pltpu.VMEM((_M,), jnp.float32),
            pltpu.VMEM((_M,), jnp.float32),
            pltpu.VMEM((_QPW,), jnp.float32),
            pltpu.VMEM((_QPW,), jnp.float32),
            pltpu.VMEM((_QPW,), jnp.float32),
            pltpu.VMEM((_QPW,), jnp.float32),
            pltpu.VMEM((_QPW,), jnp.float32),
            pltpu.VMEM((_QPW,), jnp.float32),
            pltpu.VMEM((_Q, _SEG, _L), jnp.float32),
            pltpu.VMEM((_QPW, _L), jnp.float32),
            pltpu.VMEM((_QPW, _L), jnp.int32),
        ],
    )
    return f(qx, qy, qz, qbx, qby, qbz, px, py, pz, pxd, pyd, pzd)


def kernel(queries, points):
    q = queries[0]
    p = points[0]
    qb = lax.reduce_precision(q, 8, 7)
    pb = lax.reduce_precision(p, 8, 7)
    pd = pb * np.float32(-2.0)
    d2_16, idx_16 = _sc_knn(
        q[:, 0], q[:, 1], q[:, 2], qb[:, 0], qb[:, 1], qb[:, 2],
        p[:, 0], p[:, 1], p[:, 2], pd[:, 0], pd[:, 1], pd[:, 2])
    d2k = d2_16[:, :_K]
    idx = idx_16[:, :_K]
    valid = d2k <= (_RADIUS * _RADIUS)
    indices = jnp.where(valid, idx, -1)
    dist = jnp.sqrt(d2k)
    weights = 1.0 / (dist + 1e-7)
    weights = weights / jnp.sum(weights, axis=-1, keepdims=True)
    return (indices[None].astype(jnp.int64), weights[None], dist[None])


# speedup vs baseline: 1.9385x; 1.0356x over previous
import jax
import jax.numpy as jnp
import numpy as np
from jax import lax
from jax.experimental import pallas as pl
from jax.experimental.pallas import tpu as pltpu
from jax.experimental.pallas import tpu_sc as plsc

_K = 8
_RADIUS = 100.0
_N = 8192
_M = 16384
_NW = 32
_QPW = _N // _NW
_L = 16
_MV = _M // _L
_SEGV = 8
_SEG = _MV // _SEGV
_Q = 4
_BIG = np.float32(3.0e38)
_MU = np.float32(1e-3)


def _knn_body(qx_h, qy_h, qz_h, qbx_h, qby_h, qbz_h,
              px_h, py_h, pz_h, pxd_h, pyd_h, pzd_h,
              od2_h, oidx_h,
              bx, by, bz, p2b, qxv, qyv, qzv, qbxv, qbyv, qbzv,
              segm, quadm, od2v, oidxv):
    wid = lax.axis_index("s") * 2 + lax.axis_index("c")
    base = wid * _QPW

    pltpu.sync_copy(px_h, bx)
    pltpu.sync_copy(py_h, by)
    pltpu.sync_copy(pz_h, bz)

    def p2_step(j, _):
        pxv = bx[pl.ds(j * _L, _L)]
        pyv = by[pl.ds(j * _L, _L)]
        pzv = bz[pl.ds(j * _L, _L)]
        p2b[pl.ds(j * _L, _L)] = (pxv * pxv + pyv * pyv) + pzv * pzv
        return 0

    lax.fori_loop(0, _MV, p2_step, 0, unroll=4)

    pltpu.sync_copy(pxd_h, bx)
    pltpu.sync_copy(pyd_h, by)
    pltpu.sync_copy(pzd_h, bz)

    pltpu.sync_copy(qx_h.at[pl.ds(base, _QPW)], qxv)
    pltpu.sync_copy(qy_h.at[pl.ds(base, _QPW)], qyv)
    pltpu.sync_copy(qz_h.at[pl.ds(base, _QPW)], qzv)
    pltpu.sync_copy(qbx_h.at[pl.ds(base, _QPW)], qbxv)
    pltpu.sync_copy(qby_h.at[pl.ds(base, _QPW)], qbyv)
    pltpu.sync_copy(qbz_h.at[pl.ds(base, _QPW)], qbzv)

    lane = lax.iota(jnp.int32, _L)
    big_vec = jnp.full((_L,), _BIG, jnp.float32)
    zero_idx = jnp.zeros((_L,), jnp.int32)

    def _qp2_at(off, qbx, qby, qbz):
        pxv = bx[pl.ds(off, _L)]
        pyv = by[pl.ds(off, _L)]
        pzv = bz[pl.ds(off, _L)]
        return (qbx * pxv + qby * pyv) + qbz * pzv

    def _sprime_at(off, qbx, qby, qbz):
        p2v = p2b[pl.ds(off, _L)]
        return p2v + _qp2_at(off, qbx, qby, qbz)

    def _d2_at(off, q2, qbx, qby, qbz):
        p2v = p2b[pl.ds(off, _L)]
        return jnp.maximum((q2 + p2v) + _qp2_at(off, qbx, qby, qbz), 0.0)

    def _any(mask):
        return plsc.all_reduce_population_count(mask)[0] > 0

    def per_block(blk, _):
        q0 = blk * _L
        qxg = qxv[pl.ds(q0, _L)]
        qyg = qyv[pl.ds(q0, _L)]
        qzg = qzv[pl.ds(q0, _L)]
        qbxg = qbxv[pl.ds(q0, _L)]
        qbyg = qbyv[pl.ds(q0, _L)]
        qbzg = qbzv[pl.ds(q0, _L)]
        for sub in range(_L // _Q):
            qs = []
            for k in range(_Q):
                j = sub * _Q + k
                qx, qy, qz = qxg[j], qyg[j], qzg[j]
                q2 = (qx * qx + qy * qy) + qz * qz
                qs.append((q2, qbxg[j], qbyg[j], qbzg[j]))

            def quad_min(qd, gacc):
                mq = [big_vec] * _Q
                for si in range(4):
                    s = qd * 4 + si
                    ms = [big_vec] * _Q
                    for v in range(_SEGV):
                        off = s * (_SEGV * _L) + v * _L
                        for k in range(_Q):
                            _, qbx, qby, qbz = qs[k]
                            ms[k] = jnp.minimum(
                                ms[k], _sprime_at(off, qbx, qby, qbz))
                    for k in range(_Q):
                        segm[k, s, :] = ms[k]
                        mq[k] = jnp.minimum(mq[k], ms[k])
                for k in range(_Q):
                    quadm[k, qd, :] = mq[k]
                return tuple(jnp.minimum(gacc[k], mq[k])
                             for k in range(_Q))

            gs = lax.fori_loop(0, _SEG // 4, quad_min, (big_vec,) * _Q)

            for k in range(_Q):
                q2, qbx, qby, qbz = qs[k]
                t = jnp.sort(gs[k])[_K - 1]
                ts = t + (_MU + _MU)
                b = (t + q2) + _MU

                def merge(args):
                    R, Ri, mask, d2, ci = args
                    C = jnp.where(mask, d2, _BIG)
                    Cs, Cis = plsc.sort_key_val(C, ci, descending=True)
                    take = Cs < R
                    Lo = jnp.where(take, Cs, R)
                    Loi = jnp.where(take, Cis, Ri)
                    R2, Ri2 = plsc.sort_key_val(Lo, Loi)
                    return (R2, Ri2)

                def rescan(s, R, Ri, k=k):
                    for v in range(_SEGV):
                        off = s * (_SEGV * _L) + v * _L
                        d2 = _d2_at(off, q2, qbx, qby, qbz)
                        mask = d2 <= b
                        ci = off + lane
                        R, Ri = lax.cond(
                            _any(mask), merge,
                            lambda a: (a[0], a[1]),
                            (R, Ri, mask, d2, ci))
                    return R, Ri

                def quad_scan(qd, carry, k=k):
                    R, Ri = carry

                    def check_quad(args, k=k):
                        def seg_step(si, carry):
                            R, Ri = carry
                            s = qd * 4 + si
                            m = segm[k, s, :]
                            return lax.cond(
                                _any(m <= ts),
                                lambda a: rescan(s, a[0], a[1]),
                                lambda a: a, (R, Ri))

                        return lax.fori_loop(0, 4, seg_step, args)

                    return lax.cond(
                        _any(quadm[k, qd, :] <= ts),
                        check_quad, lambda a: a, (R, Ri))

                R, Ri = lax.fori_loop(0, _SEG // 4, quad_scan,
                                      (big_vec, zero_idx))
                qi = q0 + sub * _Q + k
                od2v[qi, :] = R
                oidxv[qi, :] = Ri
        return 0

    lax.fori_loop(0, _QPW // _L, per_block, 0)

    pltpu.sync_copy(od2v, od2_h.at[pl.ds(base, _QPW)])
    pltpu.sync_copy(oidxv, oidx_h.at[pl.ds(base, _QPW)])


@jax.jit
def _sc_knn(qx, qy, qz, qbx, qby, qbz, px, py, pz, pxd, pyd, pzd):
    mesh = plsc.VectorSubcoreMesh(core_axis_name="c", subcore_axis_name="s")
    f = pl.kernel(
        _knn_body,
        mesh=mesh,
        compiler_params=pltpu.CompilerParams(
            needs_layout_passes=False, use_tc_tiling_on_sc=False),
        out_type=[
            jax.ShapeDtypeStruct((_N, _L), jnp.float32),
            jax.ShapeDtypeStruct((_N, _L), jnp.int32),
        ],
        scratch_types=[
            pltpu.VMEM((_M,), jnp.float32),
            pltpu.VMEM((_M,), jnp.float32),
            pltpu.VMEM((_M,), jnp.float32),
            pltpu.VMEM((_M,), jnp.float32),
            pltpu.VMEM((_QPW,), jnp.float32),
            pltpu.VMEM((_QPW,), jnp.float32),
            pltpu.VMEM((_QPW,), jnp.float32),
            pltpu.VMEM((_QPW,), jnp.float32),
            pltpu.VMEM((_QPW,), jnp.float32),
            pltpu.VMEM((_QPW,), jnp.float32),
            pltpu.VMEM((_Q, _SEG, _L), jnp.float32),
            pltpu.VMEM((_Q, _SEG // 4, _L), jnp.float32),
            pltpu.VMEM((_QPW, _L), jnp.float32),
            pltpu.VMEM((_QPW, _L), jnp.int32),
        ],
    )
    return f(qx, qy, qz, qbx, qby, qbz, px, py, pz, pxd, pyd, pzd)


def kernel(queries, points):
    q = queries[0]
    p = points[0]
    qb = lax.reduce_precision(q, 8, 7)
    pb = lax.reduce_precision(p, 8, 7)
    pd = pb * np.float32(-2.0)
    d2_16, idx_16 = _sc_knn(
        q[:, 0], q[:, 1], q[:, 2], qb[:, 0], qb[:, 1], qb[:, 2],
        p[:, 0], p[:, 1], p[:, 2], pd[:, 0], pd[:, 1], pd[:, 2])
    d2k = d2_16[:, :_K]
    idx = idx_16[:, :_K]
    valid = d2k <= (_RADIUS * _RADIUS)
    indices = jnp.where(valid, idx, -1)
    dist = jnp.sqrt(d2k)
    weights = 1.0 / (dist + 1e-7)
    weights = weights / jnp.sum(weights, axis=-1, keepdims=True)
    return (indices[None].astype(jnp.int64), weights[None], dist[None])


# speedup vs baseline: 1.9815x; 1.0222x over previous
import jax
import jax.numpy as jnp
import numpy as np
from jax import lax
from jax.experimental import pallas as pl
from jax.experimental.pallas import tpu as pltpu
from jax.experimental.pallas import tpu_sc as plsc

_K = 8
_RADIUS = 100.0
_N = 8192
_M = 16384
_NW = 32
_QPW = _N // _NW
_L = 16
_MV = _M // _L
_SEGV = 8
_SEG = _MV // _SEGV
_Q = 8
_BIG = np.float32(3.0e38)
_MU = np.float32(1e-3)


def _knn_body(qx_h, qy_h, qz_h, qbx_h, qby_h, qbz_h,
              px_h, py_h, pz_h, pxd_h, pyd_h, pzd_h,
              od2_h, oidx_h,
              bx, by, bz, p2b, qxv, qyv, qzv, qbxv, qbyv, qbzv,
              segm, quadm, od2v, oidxv):
    wid = lax.axis_index("s") * 2 + lax.axis_index("c")
    base = wid * _QPW

    pltpu.sync_copy(px_h, bx)
    pltpu.sync_copy(py_h, by)
    pltpu.sync_copy(pz_h, bz)

    def p2_step(j, _):
        pxv = bx[pl.ds(j * _L, _L)]
        pyv = by[pl.ds(j * _L, _L)]
        pzv = bz[pl.ds(j * _L, _L)]
        p2b[pl.ds(j * _L, _L)] = (pxv * pxv + pyv * pyv) + pzv * pzv
        return 0

    lax.fori_loop(0, _MV, p2_step, 0, unroll=4)

    pltpu.sync_copy(pxd_h, bx)
    pltpu.sync_copy(pyd_h, by)
    pltpu.sync_copy(pzd_h, bz)

    pltpu.sync_copy(qx_h.at[pl.ds(base, _QPW)], qxv)
    pltpu.sync_copy(qy_h.at[pl.ds(base, _QPW)], qyv)
    pltpu.sync_copy(qz_h.at[pl.ds(base, _QPW)], qzv)
    pltpu.sync_copy(qbx_h.at[pl.ds(base, _QPW)], qbxv)
    pltpu.sync_copy(qby_h.at[pl.ds(base, _QPW)], qbyv)
    pltpu.sync_copy(qbz_h.at[pl.ds(base, _QPW)], qbzv)

    lane = lax.iota(jnp.int32, _L)
    big_vec = jnp.full((_L,), _BIG, jnp.float32)
    zero_idx = jnp.zeros((_L,), jnp.int32)

    def _qp2_at(off, qbx, qby, qbz):
        pxv = bx[pl.ds(off, _L)]
        pyv = by[pl.ds(off, _L)]
        pzv = bz[pl.ds(off, _L)]
        return (qbx * pxv + qby * pyv) + qbz * pzv

    def _sprime_at(off, qbx, qby, qbz):
        p2v = p2b[pl.ds(off, _L)]
        return p2v + _qp2_at(off, qbx, qby, qbz)

    def _d2_at(off, q2, qbx, qby, qbz):
        p2v = p2b[pl.ds(off, _L)]
        return jnp.maximum((q2 + p2v) + _qp2_at(off, qbx, qby, qbz), 0.0)

    def _any(mask):
        return plsc.all_reduce_population_count(mask)[0] > 0

    def per_block(blk, _):
        q0 = blk * _L
        qxg = qxv[pl.ds(q0, _L)]
        qyg = qyv[pl.ds(q0, _L)]
        qzg = qzv[pl.ds(q0, _L)]
        qbxg = qbxv[pl.ds(q0, _L)]
        qbyg = qbyv[pl.ds(q0, _L)]
        qbzg = qbzv[pl.ds(q0, _L)]
        for sub in range(_L // _Q):
            qs = []
            for k in range(_Q):
                j = sub * _Q + k
                qx, qy, qz = qxg[j], qyg[j], qzg[j]
                q2 = (qx * qx + qy * qy) + qz * qz
                qs.append((q2, qbxg[j], qbyg[j], qbzg[j]))

            def quad_min(qd, gacc):
                mq = [big_vec] * _Q
                for si in range(4):
                    s = qd * 4 + si
                    ms = [big_vec] * _Q
                    for v in range(_SEGV):
                        off = s * (_SEGV * _L) + v * _L
                        for k in range(_Q):
                            _, qbx, qby, qbz = qs[k]
                            ms[k] = jnp.minimum(
                                ms[k], _sprime_at(off, qbx, qby, qbz))
                    for k in range(_Q):
                        segm[k, s, :] = ms[k]
                        mq[k] = jnp.minimum(mq[k], ms[k])
                for k in range(_Q):
                    quadm[k, qd, :] = mq[k]
                return tuple(jnp.minimum(gacc[k], mq[k])
                             for k in range(_Q))

            gs = lax.fori_loop(0, _SEG // 4, quad_min, (big_vec,) * _Q)

            for k in range(_Q):
                q2, qbx, qby, qbz = qs[k]
                t = jnp.sort(gs[k])[_K - 1]
                ts = t + (_MU + _MU)
                b = (t + q2) + _MU

                def merge(args):
                    R, Ri, mask, d2, ci = args
                    C = jnp.where(mask, d2, _BIG)
                    Cs, Cis = plsc.sort_key_val(C, ci, descending=True)
                    take = Cs < R
                    Lo = jnp.where(take, Cs, R)
                    Loi = jnp.where(take, Cis, Ri)
                    R2, Ri2 = plsc.sort_key_val(Lo, Loi)
                    return (R2, Ri2)

                def rescan(s, R, Ri, k=k):
                    for v in range(_SEGV):
                        off = s * (_SEGV * _L) + v * _L
                        d2 = _d2_at(off, q2, qbx, qby, qbz)
                        mask = d2 <= b
                        ci = off + lane
                        R, Ri = lax.cond(
                            _any(mask), merge,
                            lambda a: (a[0], a[1]),
                            (R, Ri, mask, d2, ci))
                    return R, Ri

                def quad_scan(qd, carry, k=k):
                    R, Ri = carry

                    def check_quad(args, k=k):
                        def seg_step(si, carry):
                            R, Ri = carry
                            s = qd * 4 + si
                            m = segm[k, s, :]
                            return lax.cond(
                                _any(m <= ts),
                                lambda a: rescan(s, a[0], a[1]),
                                lambda a: a, (R, Ri))

                        return lax.fori_loop(0, 4, seg_step, args)

                    return lax.cond(
                        _any(quadm[k, qd, :] <= ts),
                        check_quad, lambda a: a, (R, Ri))

                R, Ri = lax.fori_loop(0, _SEG // 4, quad_scan,
                                      (big_vec, zero_idx))
                qi = q0 + sub * _Q + k
                od2v[qi, :] = R
                oidxv[qi, :] = Ri
        return 0

    lax.fori_loop(0, _QPW // _L, per_block, 0)

    pltpu.sync_copy(od2v, od2_h.at[pl.ds(base, _QPW)])
    pltpu.sync_copy(oidxv, oidx_h.at[pl.ds(base, _QPW)])


@jax.jit
def _sc_knn(qx, qy, qz, qbx, qby, qbz, px, py, pz, pxd, pyd, pzd):
    mesh = plsc.VectorSubcoreMesh(core_axis_name="c", subcore_axis_name="s")
    f = pl.kernel(
        _knn_body,
        mesh=mesh,
        compiler_params=pltpu.CompilerParams(
            needs_layout_passes=False, use_tc_tiling_on_sc=False),
        out_type=[
            jax.ShapeDtypeStruct((_N, _L), jnp.float32),
            jax.ShapeDtypeStruct((_N, _L), jnp.int32),
        ],
        scratch_types=[
            pltpu.VMEM((_M,), jnp.float32),
            pltpu.VMEM((_M,), jnp.float32),
            pltpu.VMEM((_M,), jnp.float32),
            pltpu.VMEM((_M,), jnp.float32),
            pltpu.VMEM((_QPW,), jnp.float32),
            pltpu.VMEM((_QPW,), jnp.float32),
            pltpu.VMEM((_QPW,), jnp.float32),
            pltpu.VMEM((_QPW,), jnp.float32),
            pltpu.VMEM((_QPW,), jnp.float32),
            pltpu.VMEM((_QPW,), jnp.float32),
            pltpu.VMEM((_Q, _SEG, _L), jnp.float32),
            pltpu.VMEM((_Q, _SEG // 4, _L), jnp.float32),
            pltpu.VMEM((_QPW, _L), jnp.float32),
            pltpu.VMEM((_QPW, _L), jnp.int32),
        ],
    )
    return f(qx, qy, qz, qbx, qby, qbz, px, py, pz, pxd, pyd, pzd)


def kernel(queries, points):
    q = queries[0]
    p = points[0]
    qb = lax.reduce_precision(q, 8, 7)
    pb = lax.reduce_precision(p, 8, 7)
    pd = pb * np.float32(-2.0)
    d2_16, idx_16 = _sc_knn(
        q[:, 0], q[:, 1], q[:, 2], qb[:, 0], qb[:, 1], qb[:, 2],
        p[:, 0], p[:, 1], p[:, 2], pd[:, 0], pd[:, 1], pd[:, 2])
    d2k = d2_16[:, :_K]
    idx = idx_16[:, :_K]
    valid = d2k <= (_RADIUS * _RADIUS)
    indices = jnp.where(valid, idx, -1)
    dist = jnp.sqrt(d2k)
    weights = 1.0 / (dist + 1e-7)
    weights = weights / jnp.sum(weights, axis=-1, keepdims=True)
    return (indices[None].astype(jnp.int64), weights[None], dist[None])
